# baseline (speedup 1.0000x reference)
import jax
import jax.numpy as jnp
from jax.experimental import pallas as pl
from jax.experimental.pallas import tpu as pltpu

_B = 5000
_N = 20000
_C = 256
_CE = 384
_BT = 1024
_H = 512
_NT = 2048
_NJ = 5
_NN = 10
_BP = _BT * _NJ
_NP = _NT * _NN
_T = _NN * _NJ
_CHUNK = 256
_NCH = _NT // _CHUNK


def _roi_body(boxes_ref, coords_ref, fhep_ref, flep_ref, fhec_ref, flec_ref,
              mask_ref, bf_ref, m0_ref, m1_ref, acc_ref):
    u = pl.program_id(0)
    iN = u // _NJ
    j = u % _NJ
    p = jnp.maximum(u - 1, 0)
    iNp = p // _NJ
    jp = p % _NJ

    @pl.when(u == 0)
    def _zero_m1():
        m1_ref[...] = jnp.zeros_like(m1_ref)

    @pl.when(iN == 0)
    def _zero_acc():
        acc_ref[pl.ds(j * _BT, _BT), :] = jnp.zeros((_BT, _CE), jnp.float32)

    x0 = boxes_ref[:, 0:1]
    y0 = boxes_ref[:, 1:2]
    x1 = boxes_ref[:, 2:3]
    y1 = boxes_ref[:, 3:4]
    af = boxes_ref[:, 4:5]

    def mask_chunk(h, m_scr, k):
        r = slice(h * _H, (h + 1) * _H)
        sl = pl.ds(k * _CHUNK, _CHUNK)
        x = coords_ref[0:1, sl]
        y = coords_ref[1:2, sl]
        s = coords_ref[2:3, sl]
        inside = ((x0[r] <= x) & (x < x1[r]) & (y0[r] <= y) & (y < y1[r])
                  & (s == af[r]))
        mask_ref[r, sl] = inside
        m_scr[:, sl] = jnp.where(inside, jnp.float32(1),
                                 jnp.float32(0)).astype(jnp.bfloat16)

    r1p = pl.ds(jp * _BT + _H, _H)

    acc_ref[r1p, :] += jnp.dot(m1_ref[...], fhep_ref[...],
                               preferred_element_type=jnp.float32)
    for k in range(_NCH // 2):
        mask_chunk(0, m0_ref, k)
    acc_ref[r1p, :_C] += jnp.dot(m1_ref[...], flep_ref[...],
                                 preferred_element_type=jnp.float32)
    for k in range(_NCH // 2, _NCH):
        mask_chunk(0, m0_ref, k)

    @pl.when(u < _T)
    def _h0_dots_h1_mask():
        r0 = pl.ds(j * _BT, _H)
        acc_ref[r0, :] += jnp.dot(m0_ref[...], fhec_ref[...],
                                  preferred_element_type=jnp.float32)
        for k in range(_NCH // 2):
            mask_chunk(1, m1_ref, k)
        acc_ref[r0, :_C] += jnp.dot(m0_ref[...], flec_ref[...],
                                    preferred_element_type=jnp.float32)
        for k in range(_NCH // 2, _NCH):
            mask_chunk(1, m1_ref, k)

    @pl.when(iNp == _NN - 1)
    def _fin():
        rows = pl.ds(jp * _BT, _BT)
        tot = acc_ref[rows, :_C]
        c = jnp.maximum(acc_ref[rows, _C:_C + 1], 1.0)
        bf_ref[...] = tot / c


def kernel(coords, features, bbox_tensor, bbox_sample_association):
    af = bbox_sample_association.astype(jnp.float32)
    boxes = jnp.concatenate(
        [bbox_tensor[:, 0, :], bbox_tensor[:, 1, :], af[:, None]], axis=-1)
    boxes_p = jnp.pad(boxes, ((0, _BP - _B), (0, 8 - 5)),
                      constant_values=1e30)
    coords_t = jnp.pad(coords.T, ((0, 8 - 3), (0, _NP - _N)),
                       constant_values=-1e30)
    fh = features.astype(jnp.bfloat16)
    fl = (features - fh.astype(jnp.float32)).astype(jnp.bfloat16)
    fhe = jnp.zeros((_NP, _CE), dtype=jnp.bfloat16)
    fhe = fhe.at[:_N, :_C].set(fh)
    fhe = fhe.at[:_N, _C].set(jnp.bfloat16(1.0))
    fle = jnp.pad(fl, ((0, _NP - _N), (0, 0)))

    def _bx(u):
        return (jnp.where(u >= _T, _NJ - 1, u % _NJ), 0)

    def _cx(u):
        return (0, jnp.minimum(u // _NJ, _NN - 1))

    def _mx(u):
        return (jnp.where(u >= _T, _NJ - 1, u % _NJ),
                jnp.minimum(u // _NJ, _NN - 1))

    def _bfx(u):
        return (jnp.where(u == 0, 0, (u - 1) % _NJ), 0)

    def _fxp(u):
        return (jnp.maximum(u - 1, 0) // _NJ, 0)

    def _fxc(u):
        return (jnp.minimum(u // _NJ, _NN - 1), 0)

    is_inside, box_features = pl.pallas_call(
        _roi_body,
        grid=(_T + 1,),
        in_specs=[
            pl.BlockSpec((_BT, 8), _bx),
            pl.BlockSpec((8, _NT), _cx),
            pl.BlockSpec((_NT, _CE), _fxp),
            pl.BlockSpec((_NT, _C), _fxp),
            pl.BlockSpec((_NT, _CE), _fxc),
            pl.BlockSpec((_NT, _C), _fxc),
        ],
        out_specs=[
            pl.BlockSpec((_BT, _NT), _mx),
            pl.BlockSpec((_BT, _C), _bfx),
        ],
        out_shape=[
            jax.ShapeDtypeStruct((_B, _N), jnp.bool_),
            jax.ShapeDtypeStruct((_B, _C), jnp.float32),
        ],
        scratch_shapes=[
            pltpu.VMEM((_H, _NT), jnp.bfloat16),
            pltpu.VMEM((_H, _NT), jnp.bfloat16),
            pltpu.VMEM((_BP, _CE), jnp.float32),
        ],
    )(boxes_p, coords_t, fhe, fle, fhe, fle)
    return (box_features, is_inside)


# speedup vs baseline: 1.0951x; 1.0951x over previous
import jax
import jax.numpy as jnp
from jax.experimental import pallas as pl
from jax.experimental.pallas import tpu as pltpu

_B = 5000
_N = 20000
_C = 256
_CE = 384
_BT = 1024
_H = 512
_NT = 2048
_NJ = 5
_NN = 10
_BP = _BT * _NJ
_NP = _NT * _NN
_T = _NN * _NJ
_CHUNK = 256
_NCH = _NT // _CHUNK


def _roi_body(boxes_ref, coords_ref, fhep_ref, flep_ref, fhec_ref, flec_ref,
              mask_ref, bf_ref, m0_ref, m1_ref, acc_ref):
    u = pl.program_id(0)
    iN = u // _NJ
    j = u % _NJ
    p = jnp.maximum(u - 1, 0)
    iNp = p // _NJ
    jp = p % _NJ

    @pl.when(u == 0)
    def _zero_m1():
        m1_ref[...] = jnp.zeros_like(m1_ref)

    @pl.when(iN == 0)
    def _zero_acc():
        acc_ref[pl.ds(j * _BT, _BT), :] = jnp.zeros((_BT, _CE), jnp.float32)

    x0 = boxes_ref[:, 0:1]
    y0 = boxes_ref[:, 1:2]
    x1 = boxes_ref[:, 2:3]
    y1 = boxes_ref[:, 3:4]
    af = boxes_ref[:, 4:5]

    def mask_chunk(h, m_scr, k):
        r = slice(h * _H, (h + 1) * _H)
        sl = pl.ds(k * _CHUNK, _CHUNK)
        x = coords_ref[0:1, sl]
        y = coords_ref[1:2, sl]
        s = coords_ref[2:3, sl]
        inside = ((x0[r] <= x) & (x < x1[r]) & (y0[r] <= y) & (y < y1[r])
                  & (s == af[r]))
        mf = jnp.where(inside, jnp.float32(1), jnp.float32(0))
        mask_ref[r, sl] = mf.astype(jnp.int8)
        m_scr[:, sl] = mf.astype(jnp.bfloat16)

    r1p = pl.ds(jp * _BT + _H, _H)

    acc_ref[r1p, :] += jnp.dot(m1_ref[...], fhep_ref[...],
                               preferred_element_type=jnp.float32)
    for k in range(_NCH // 2):
        mask_chunk(0, m0_ref, k)
    acc_ref[r1p, :_C] += jnp.dot(m1_ref[...], flep_ref[...],
                                 preferred_element_type=jnp.float32)
    for k in range(_NCH // 2, _NCH):
        mask_chunk(0, m0_ref, k)

    @pl.when(u < _T)
    def _h0_dots_h1_mask():
        r0 = pl.ds(j * _BT, _H)
        acc_ref[r0, :] += jnp.dot(m0_ref[...], fhec_ref[...],
                                  preferred_element_type=jnp.float32)
        for k in range(_NCH // 2):
            mask_chunk(1, m1_ref, k)
        acc_ref[r0, :_C] += jnp.dot(m0_ref[...], flec_ref[...],
                                    preferred_element_type=jnp.float32)
        for k in range(_NCH // 2, _NCH):
            mask_chunk(1, m1_ref, k)

    @pl.when(iNp == _NN - 1)
    def _fin():
        rows = pl.ds(jp * _BT, _BT)
        tot = acc_ref[rows, :_C]
        c = jnp.maximum(acc_ref[rows, _C:_C + 1], 1.0)
        bf_ref[...] = tot / c


def kernel(coords, features, bbox_tensor, bbox_sample_association):
    af = bbox_sample_association.astype(jnp.float32)
    boxes = jnp.concatenate(
        [bbox_tensor[:, 0, :], bbox_tensor[:, 1, :], af[:, None]], axis=-1)
    boxes_p = jnp.pad(boxes, ((0, _BP - _B), (0, 8 - 5)),
                      constant_values=1e30)
    coords_t = jnp.pad(coords.T, ((0, 8 - 3), (0, _NP - _N)),
                       constant_values=-1e30)
    fh = features.astype(jnp.bfloat16)
    fl = (features - fh.astype(jnp.float32)).astype(jnp.bfloat16)
    fhe = jnp.zeros((_NP, _CE), dtype=jnp.bfloat16)
    fhe = fhe.at[:_N, :_C].set(fh)
    fhe = fhe.at[:_N, _C].set(jnp.bfloat16(1.0))
    fle = jnp.pad(fl, ((0, _NP - _N), (0, 0)))

    def _bx(u):
        return (jnp.where(u >= _T, _NJ - 1, u % _NJ), 0)

    def _cx(u):
        return (0, jnp.minimum(u // _NJ, _NN - 1))

    def _mx(u):
        return (jnp.where(u >= _T, _NJ - 1, u % _NJ),
                jnp.minimum(u // _NJ, _NN - 1))

    def _bfx(u):
        return (jnp.where(u == 0, 0, (u - 1) % _NJ), 0)

    def _fxp(u):
        return (jnp.maximum(u - 1, 0) // _NJ, 0)

    def _fxc(u):
        return (jnp.minimum(u // _NJ, _NN - 1), 0)

    is_inside, box_features = pl.pallas_call(
        _roi_body,
        grid=(_T + 1,),
        in_specs=[
            pl.BlockSpec((_BT, 8), _bx),
            pl.BlockSpec((8, _NT), _cx),
            pl.BlockSpec((_NT, _CE), _fxp),
            pl.BlockSpec((_NT, _C), _fxp),
            pl.BlockSpec((_NT, _CE), _fxc),
            pl.BlockSpec((_NT, _C), _fxc),
        ],
        out_specs=[
            pl.BlockSpec((_BT, _NT), _mx),
            pl.BlockSpec((_BT, _C), _bfx),
        ],
        out_shape=[
            jax.ShapeDtypeStruct((_B, _N), jnp.int8),
            jax.ShapeDtypeStruct((_B, _C), jnp.float32),
        ],
        scratch_shapes=[
            pltpu.VMEM((_H, _NT), jnp.bfloat16),
            pltpu.VMEM((_H, _NT), jnp.bfloat16),
            pltpu.VMEM((_BP, _CE), jnp.float32),
        ],
    )(boxes_p, coords_t, fhe, fle, fhe, fle)
    return (box_features, is_inside.view(jnp.bool_))


# speedup vs baseline: 1.6227x; 1.4817x over previous
import jax
import jax.numpy as jnp
from jax.experimental import pallas as pl
from jax.experimental.pallas import tpu as pltpu

_B = 5000
_N = 20000
_C = 256
_CE = 384
_BT = 512
_NT = 2048
_NB = 10
_NN = 10
_BP = _BT * _NB
_NP = _NT * _NN
_SCALE = 32768.0
_SHIFT = 1 << 24


def _roi_body(boxes_ref, coords_ref, fhe_ref, mask_ref, bf_ref, acc_ref):
    i_n = pl.program_id(0)
    i_b = pl.program_id(1)

    xq = coords_ref[0:1, :]
    yq = coords_ref[1:2, :]
    x0 = boxes_ref[:, 0:1]
    x1 = boxes_ref[:, 1:2]
    y0 = boxes_ref[:, 2:3]
    y1 = boxes_ref[:, 3:4]

    inside = (x0 <= xq) & (xq <= x1) & (y0 <= yq) & (yq <= y1)
    mf = jnp.where(inside, jnp.float32(1), jnp.float32(0))
    mask_ref[...] = mf.astype(jnp.int8)
    m = mf.astype(jnp.bfloat16)
    part = jnp.dot(m, fhe_ref[...], preferred_element_type=jnp.float32)

    rows = pl.ds(i_b * _BT, _BT)

    @pl.when(i_n == 0)
    def _init():
        acc_ref[rows, :] = part

    @pl.when(i_n > 0)
    def _acc():
        acc_ref[rows, :] = acc_ref[rows, :] + part

    @pl.when(i_n == _NN - 1)
    def _fin():
        tot = acc_ref[rows, :_C]
        c = jnp.maximum(acc_ref[rows, _C:_C + 1], 1.0)
        bf_ref[...] = tot / c


def kernel(coords, features, bbox_tensor, bbox_sample_association):
    f32, i32 = jnp.float32, jnp.int32

    xk = (coords[:, 0] * _SCALE).astype(i32)
    yk = (coords[:, 1] * _SCALE).astype(i32)
    sk = coords[:, 2].astype(i32)
    xq = xk + sk * _SHIFT
    a = bbox_sample_association.astype(i32)
    x0 = jnp.ceil(bbox_tensor[:, 0, 0] * _SCALE).astype(i32) + a * _SHIFT
    y0 = jnp.ceil(bbox_tensor[:, 0, 1] * _SCALE).astype(i32)
    x1 = (jnp.minimum(jnp.ceil(bbox_tensor[:, 1, 0] * _SCALE).astype(i32) - 1,
                      _SHIFT - 1) + a * _SHIFT)
    y1 = jnp.ceil(bbox_tensor[:, 1, 1] * _SCALE).astype(i32) - 1

    boxes_p = jnp.pad(jnp.stack([x0, x1, y0, y1], axis=-1),
                      ((0, _BP - _B), (0, 8 - 4)),
                      constant_values=jnp.iinfo(jnp.int32).max)
    boxes_p = boxes_p.at[_B:, 1].set(0)
    coords_q = jnp.pad(jnp.stack([xq, yk], axis=0),
                       ((0, 8 - 2), (0, _NP - _N)),
                       constant_values=-(1 << 30))

    fh = features.astype(jnp.bfloat16)
    fhe = jnp.zeros((_NP, _CE), dtype=jnp.bfloat16)
    fhe = fhe.at[:_N, :_C].set(fh)
    fhe = fhe.at[:_N, _C].set(jnp.bfloat16(1.0))

    grid = (_NN, _NB)
    is_inside, box_features = pl.pallas_call(
        _roi_body,
        grid=grid,
        in_specs=[
            pl.BlockSpec((_BT, 8), lambda i_n, i_b: (i_b, 0)),
            pl.BlockSpec((8, _NT), lambda i_n, i_b: (0, i_n)),
            pl.BlockSpec((_NT, _CE), lambda i_n, i_b: (i_n, 0)),
        ],
        out_specs=[
            pl.BlockSpec((_BT, _NT), lambda i_n, i_b: (i_b, i_n)),
            pl.BlockSpec((_BT, _C), lambda i_n, i_b: (i_b, 0)),
        ],
        out_shape=[
            jax.ShapeDtypeStruct((_B, _N), jnp.int8),
            jax.ShapeDtypeStruct((_B, _C), jnp.float32),
        ],
        scratch_shapes=[
            pltpu.VMEM((_BP, _CE), jnp.float32),
        ],
    )(boxes_p, coords_q, fhe)
    return (box_features, is_inside.view(jnp.bool_))


# speedup vs baseline: 1.7463x; 1.0761x over previous
import jax
import jax.numpy as jnp
from jax.experimental import pallas as pl
from jax.experimental.pallas import tpu as pltpu

_B = 5000
_N = 20000
_C = 256
_CE = 384
_BT = 1024
_NT = 4096
_NB = 5
_NN = 5
_BP = _BT * _NB
_NP = _NT * _NN
_SCALE = 32768.0
_SHIFT = 1 << 24


def _roi_body(boxes_ref, coords_ref, fhe_ref, mask_ref, bf_ref, acc_ref):
    i_n = pl.program_id(0)
    i_b = pl.program_id(1)

    xq = coords_ref[0:1, :]
    yq = coords_ref[1:2, :]
    x0 = boxes_ref[:, 0:1]
    x1 = boxes_ref[:, 1:2]
    y0 = boxes_ref[:, 2:3]
    y1 = boxes_ref[:, 3:4]

    inside = (x0 <= xq) & (xq <= x1) & (y0 <= yq) & (yq <= y1)
    mf = jnp.where(inside, jnp.float32(1), jnp.float32(0))
    mask_ref[...] = mf.astype(jnp.int8)
    m = mf.astype(jnp.bfloat16)
    part = jnp.dot(m, fhe_ref[...], preferred_element_type=jnp.float32)

    rows = pl.ds(i_b * _BT, _BT)

    @pl.when(i_n == 0)
    def _init():
        acc_ref[rows, :] = part

    @pl.when(i_n > 0)
    def _acc():
        acc_ref[rows, :] = acc_ref[rows, :] + part

    @pl.when(i_n == _NN - 1)
    def _fin():
        tot = acc_ref[rows, :_C]
        c = jnp.maximum(acc_ref[rows, _C:_C + 1], 1.0)
        bf_ref[...] = tot / c


def kernel(coords, features, bbox_tensor, bbox_sample_association):
    f32, i32 = jnp.float32, jnp.int32

    xk = (coords[:, 0] * _SCALE).astype(i32)
    yk = (coords[:, 1] * _SCALE).astype(i32)
    sk = coords[:, 2].astype(i32)
    xq = xk + sk * _SHIFT
    a = bbox_sample_association.astype(i32)
    x0 = jnp.ceil(bbox_tensor[:, 0, 0] * _SCALE).astype(i32) + a * _SHIFT
    y0 = jnp.ceil(bbox_tensor[:, 0, 1] * _SCALE).astype(i32)
    x1 = (jnp.minimum(jnp.ceil(bbox_tensor[:, 1, 0] * _SCALE).astype(i32) - 1,
                      _SHIFT - 1) + a * _SHIFT)
    y1 = jnp.ceil(bbox_tensor[:, 1, 1] * _SCALE).astype(i32) - 1

    boxes_p = jnp.pad(jnp.stack([x0, x1, y0, y1], axis=-1),
                      ((0, _BP - _B), (0, 8 - 4)),
                      constant_values=jnp.iinfo(jnp.int32).max)
    boxes_p = boxes_p.at[_B:, 1].set(0)
    coords_q = jnp.pad(jnp.stack([xq, yk], axis=0),
                       ((0, 8 - 2), (0, _NP - _N)),
                       constant_values=-(1 << 30))

    fh = features.astype(jnp.bfloat16)
    fhe = jnp.zeros((_NP, _CE), dtype=jnp.bfloat16)
    fhe = fhe.at[:_N, :_C].set(fh)
    fhe = fhe.at[:_N, _C].set(jnp.bfloat16(1.0))

    grid = (_NN, _NB)
    is_inside, box_features = pl.pallas_call(
        _roi_body,
        grid=grid,
        in_specs=[
            pl.BlockSpec((_BT, 8), lambda i_n, i_b: (i_b, 0)),
            pl.BlockSpec((8, _NT), lambda i_n, i_b: (0, i_n)),
            pl.BlockSpec((_NT, _CE), lambda i_n, i_b: (i_n, 0)),
        ],
        out_specs=[
            pl.BlockSpec((_BT, _NT), lambda i_n, i_b: (i_b, i_n)),
            pl.BlockSpec((_BT, _C), lambda i_n, i_b: (i_b, 0)),
        ],
        out_shape=[
            jax.ShapeDtypeStruct((_B, _N), jnp.int8),
            jax.ShapeDtypeStruct((_B, _C), jnp.float32),
        ],
        scratch_shapes=[
            pltpu.VMEM((_BP, _CE), jnp.float32),
        ],
    )(boxes_p, coords_q, fhe)
    return (box_features, is_inside.view(jnp.bool_))


# speedup vs baseline: 1.7666x; 1.0117x over previous
import jax
import jax.numpy as jnp
from jax.experimental import pallas as pl
from jax.experimental.pallas import tpu as pltpu

_B = 5000
_N = 20000
_C = 256
_CE = 384
_BT = 1024
_NT = 4096
_NB = 5
_NN = 5
_BP = _BT * _NB
_NP = _NT * _NN
_SCALE = 32768.0
_SHIFT = 1 << 24


def _roi_body(boxes_ref, coords_ref, fhe_ref, mask_ref, bf_ref, acc_ref):
    i_n = pl.program_id(0)
    i_b = pl.program_id(1)

    xq = coords_ref[0:1, :]
    yq = coords_ref[1:2, :]
    x0 = boxes_ref[:, 0:1]
    wx = boxes_ref[:, 1:2]
    y0 = boxes_ref[:, 2:3]
    wy = boxes_ref[:, 3:4]

    inside = ((xq - x0) <= wx) & ((yq - y0) <= wy)
    mf = jnp.where(inside, jnp.float32(1), jnp.float32(0))
    mask_ref[...] = mf.astype(jnp.int8)
    m = mf.astype(jnp.bfloat16)
    part = jnp.dot(m, fhe_ref[...], preferred_element_type=jnp.float32)

    rows = pl.ds(i_b * _BT, _BT)

    @pl.when(i_n == 0)
    def _init():
        acc_ref[rows, :] = part

    @pl.when(i_n > 0)
    def _acc():
        acc_ref[rows, :] = acc_ref[rows, :] + part

    @pl.when(i_n == _NN - 1)
    def _fin():
        tot = acc_ref[rows, :_C]
        c = jnp.maximum(acc_ref[rows, _C:_C + 1], 1.0)
        bf_ref[...] = tot / c


def kernel(coords, features, bbox_tensor, bbox_sample_association):
    f32, i32 = jnp.float32, jnp.int32

    xk = (coords[:, 0] * _SCALE).astype(i32)
    yk = (coords[:, 1] * _SCALE).astype(i32)
    sk = coords[:, 2].astype(i32)
    xq = xk + sk * _SHIFT
    a = bbox_sample_association.astype(i32)
    x0 = jnp.ceil(bbox_tensor[:, 0, 0] * _SCALE).astype(i32) + a * _SHIFT
    y0 = jnp.ceil(bbox_tensor[:, 0, 1] * _SCALE).astype(i32)
    x1 = (jnp.minimum(jnp.ceil(bbox_tensor[:, 1, 0] * _SCALE).astype(i32) - 1,
                      _SHIFT - 1) + a * _SHIFT)
    y1 = jnp.ceil(bbox_tensor[:, 1, 1] * _SCALE).astype(i32) - 1

    wx = x1 - x0
    wy = y1 - y0
    boxes_p = jnp.pad(jnp.stack([x0, wx, y0, wy], axis=-1),
                      ((0, _BP - _B), (0, 8 - 4)),
                      constant_values=0)
    boxes_p = boxes_p.at[_B:, 0].set(1 << 30)
    boxes_p = boxes_p.at[_B:, 2].set(1 << 30)
    boxes_p = boxes_p.astype(jnp.uint32)
    coords_q = jnp.pad(jnp.stack([xq, yk], axis=0),
                       ((0, 8 - 2), (0, _NP - _N)),
                       constant_values=-(1 << 31)).astype(jnp.uint32)

    fh = features.astype(jnp.bfloat16)
    fhe = jnp.zeros((_NP, _CE), dtype=jnp.bfloat16)
    fhe = fhe.at[:_N, :_C].set(fh)
    fhe = fhe.at[:_N, _C].set(jnp.bfloat16(1.0))

    grid = (_NN, _NB)
    is_inside, box_features = pl.pallas_call(
        _roi_body,
        grid=grid,
        in_specs=[
            pl.BlockSpec((_BT, 8), lambda i_n, i_b: (i_b, 0)),
            pl.BlockSpec((8, _NT), lambda i_n, i_b: (0, i_n)),
            pl.BlockSpec((_NT, _CE), lambda i_n, i_b: (i_n, 0)),
        ],
        out_specs=[
            pl.BlockSpec((_BT, _NT), lambda i_n, i_b: (i_b, i_n)),
            pl.BlockSpec((_BT, _C), lambda i_n, i_b: (i_b, 0)),
        ],
        out_shape=[
            jax.ShapeDtypeStruct((_B, _N), jnp.int8),
            jax.ShapeDtypeStruct((_B, _C), jnp.float32),
        ],
        scratch_shapes=[
            pltpu.VMEM((_BP, _CE), jnp.float32),
        ],
    )(boxes_p, coords_q, fhe)
    return (box_features, is_inside.view(jnp.bool_))


# speedup vs baseline: 2.0124x; 1.1391x over previous
import jax
import jax.numpy as jnp
from jax.experimental import pallas as pl
from jax.experimental.pallas import tpu as pltpu

_B = 5000
_N = 20000
_C = 256
_CE = 384
_BT = 1024
_NT = 4096
_NB = 5
_NN = 5
_BP = _BT * _NB
_NP = _NT * _NN
_SCALE = 32768.0
_SHIFT = 1 << 24


def _roi_body(boxes_ref, coords_ref, feat_ref, mask_ref, bf_ref,
              fhe_ref, acc_ref):
    i_n = pl.program_id(0)
    i_b = pl.program_id(1)

    @pl.when(i_b == 0)
    def _stage_features():
        fhe_ref[:, :_C] = feat_ref[...].astype(jnp.bfloat16)
        fhe_ref[:, _C:] = jnp.concatenate(
            [jnp.ones((_NT, 1), jnp.bfloat16),
             jnp.zeros((_NT, _CE - _C - 1), jnp.bfloat16)], axis=1)

    @pl.when((i_b == 0) & (i_n == _NN - 1))
    def _zero_tail():
        fhe_ref[pl.ds(_N - (_NN - 1) * _NT, _NN * _NT - _N), :] = jnp.zeros(
            (_NN * _NT - _N, _CE), jnp.bfloat16)

    xq = coords_ref[0:1, :]
    yq = coords_ref[1:2, :]
    x0 = boxes_ref[:, 0:1]
    wx = boxes_ref[:, 1:2]
    y0 = boxes_ref[:, 2:3]
    wy = boxes_ref[:, 3:4]

    inside = ((xq - x0) <= wx) & ((yq - y0) <= wy)
    mf = jnp.where(inside, jnp.float32(1), jnp.float32(0))
    mask_ref[...] = mf.astype(jnp.int8)
    m = mf.astype(jnp.bfloat16)
    part = jnp.dot(m, fhe_ref[...], preferred_element_type=jnp.float32)

    rows = pl.ds(i_b * _BT, _BT)

    @pl.when(i_n == 0)
    def _init():
        acc_ref[rows, :] = part

    @pl.when(i_n > 0)
    def _acc():
        acc_ref[rows, :] = acc_ref[rows, :] + part

    @pl.when(i_n == _NN - 1)
    def _fin():
        tot = acc_ref[rows, :_C]
        c = jnp.maximum(acc_ref[rows, _C:_C + 1], 1.0)
        bf_ref[...] = tot / c


def kernel(coords, features, bbox_tensor, bbox_sample_association):
    f32, i32 = jnp.float32, jnp.int32

    xk = (coords[:, 0] * _SCALE).astype(i32)
    yk = (coords[:, 1] * _SCALE).astype(i32)
    sk = coords[:, 2].astype(i32)
    xq = xk + sk * _SHIFT
    a = bbox_sample_association.astype(i32)
    x0 = jnp.ceil(bbox_tensor[:, 0, 0] * _SCALE).astype(i32) + a * _SHIFT
    y0 = jnp.ceil(bbox_tensor[:, 0, 1] * _SCALE).astype(i32)
    x1 = (jnp.minimum(jnp.ceil(bbox_tensor[:, 1, 0] * _SCALE).astype(i32) - 1,
                      _SHIFT - 1) + a * _SHIFT)
    y1 = jnp.ceil(bbox_tensor[:, 1, 1] * _SCALE).astype(i32) - 1

    wx = x1 - x0
    wy = y1 - y0
    boxes_p = jnp.pad(jnp.stack([x0, wx, y0, wy], axis=-1),
                      ((0, _BP - _B), (0, 8 - 4)),
                      constant_values=0)
    boxes_p = boxes_p.at[_B:, 0].set(1 << 30)
    boxes_p = boxes_p.at[_B:, 2].set(1 << 30)
    boxes_p = boxes_p.astype(jnp.uint32)
    coords_q = jnp.pad(jnp.stack([xq, yk], axis=0),
                       ((0, 8 - 2), (0, _NP - _N)),
                       constant_values=-(1 << 31)).astype(jnp.uint32)

    grid = (_NN, _NB)
    is_inside, box_features = pl.pallas_call(
        _roi_body,
        grid=grid,
        in_specs=[
            pl.BlockSpec((_BT, 8), lambda i_n, i_b: (i_b, 0)),
            pl.BlockSpec((8, _NT), lambda i_n, i_b: (0, i_n)),
            pl.BlockSpec((_NT, _C), lambda i_n, i_b: (i_n, 0)),
        ],
        out_specs=[
            pl.BlockSpec((_BT, _NT), lambda i_n, i_b: (i_b, i_n)),
            pl.BlockSpec((_BT, _C), lambda i_n, i_b: (i_b, 0)),
        ],
        out_shape=[
            jax.ShapeDtypeStruct((_B, _N), jnp.int8),
            jax.ShapeDtypeStruct((_B, _C), jnp.float32),
        ],
        scratch_shapes=[
            pltpu.VMEM((_NT, _CE), jnp.bfloat16),
            pltpu.VMEM((_BP, _CE), jnp.float32),
        ],
    )(boxes_p, coords_q, features)
    return (box_features, is_inside.view(jnp.bool_))


# speedup vs baseline: 2.2934x; 1.1396x over previous
import jax
import jax.numpy as jnp
from jax.experimental import pallas as pl
from jax.experimental.pallas import tpu as pltpu

_B = 5000
_N = 20000
_C = 256
_CE = 384
_BT = 1024
_NT = 4096
_NB = 5
_NN = 5
_BP = _BT * _NB
_NP = _NT * _NN
_SCALE = 32768.0
_SHIFT = 1 << 24


def _roi_body(boxes_ref, coords_ref, feat_ref, mask_ref, bf_ref,
              fhe_ref, acc_ref):
    i_n = pl.program_id(0)
    i_b = pl.program_id(1)

    @pl.when(i_b == 0)
    def _stage_features():
        fhe_ref[:, :_C] = feat_ref[...].astype(jnp.bfloat16)
        fhe_ref[:, _C:] = jnp.concatenate(
            [jnp.ones((_NT, 1), jnp.bfloat16),
             jnp.zeros((_NT, _CE - _C - 1), jnp.bfloat16)], axis=1)

    @pl.when((i_b == 0) & (i_n == _NN - 1))
    def _zero_tail():
        fhe_ref[pl.ds(_N - (_NN - 1) * _NT, _NN * _NT - _N), :] = jnp.zeros(
            (_NN * _NT - _N, _CE), jnp.bfloat16)

    xq = coords_ref[0:1, :]
    yq = coords_ref[1:2, :]
    x0 = boxes_ref[:, 0:1]
    wx = boxes_ref[:, 1:2]
    y0 = boxes_ref[:, 2:3]
    wy = boxes_ref[:, 3:4]

    cx = (xq - x0) <= wx
    cy = (yq - y0) <= wy
    mf = jnp.where(cx, jnp.where(cy, jnp.float32(1), jnp.float32(0)),
                   jnp.float32(0))
    mask_ref[...] = mf.astype(jnp.int8)
    m = mf.astype(jnp.bfloat16)
    part = jnp.dot(m, fhe_ref[...], preferred_element_type=jnp.float32)

    rows = pl.ds(i_b * _BT, _BT)

    @pl.when(i_n == 0)
    def _init():
        acc_ref[rows, :] = part

    @pl.when(i_n > 0)
    def _acc():
        acc_ref[rows, :] = acc_ref[rows, :] + part

    @pl.when(i_n == _NN - 1)
    def _fin():
        tot = acc_ref[rows, :_C]
        c = jnp.maximum(acc_ref[rows, _C:_C + 1], 1.0)
        bf_ref[...] = tot / c


def kernel(coords, features, bbox_tensor, bbox_sample_association):
    f32, i32 = jnp.float32, jnp.int32

    xk = (coords[:, 0] * _SCALE).astype(i32)
    yk = (coords[:, 1] * _SCALE).astype(i32)
    sk = coords[:, 2].astype(i32)
    xq = xk + sk * _SHIFT
    a = bbox_sample_association.astype(i32)
    x0 = jnp.ceil(bbox_tensor[:, 0, 0] * _SCALE).astype(i32) + a * _SHIFT
    y0 = jnp.ceil(bbox_tensor[:, 0, 1] * _SCALE).astype(i32)
    x1 = (jnp.minimum(jnp.ceil(bbox_tensor[:, 1, 0] * _SCALE).astype(i32) - 1,
                      _SHIFT - 1) + a * _SHIFT)
    y1 = jnp.ceil(bbox_tensor[:, 1, 1] * _SCALE).astype(i32) - 1

    wx = x1 - x0
    wy = y1 - y0
    boxes_p = jnp.pad(jnp.stack([x0, wx, y0, wy], axis=-1),
                      ((0, _BP - _B), (0, 8 - 4)),
                      constant_values=0)
    boxes_p = boxes_p.at[_B:, 0].set(1 << 30)
    boxes_p = boxes_p.at[_B:, 2].set(1 << 30)
    boxes_p = boxes_p.astype(jnp.uint32)
    coords_q = jnp.pad(jnp.stack([xq, yk], axis=0),
                       ((0, 8 - 2), (0, _NP - _N)),
                       constant_values=-(1 << 31)).astype(jnp.uint32)

    grid = (_NN, _NB)
    is_inside, box_features = pl.pallas_call(
        _roi_body,
        grid=grid,
        in_specs=[
            pl.BlockSpec((_BT, 8), lambda i_n, i_b: (i_b, 0)),
            pl.BlockSpec((8, _NT), lambda i_n, i_b: (0, i_n)),
            pl.BlockSpec((_NT, _C), lambda i_n, i_b: (i_n, 0)),
        ],
        out_specs=[
            pl.BlockSpec((_BT, _NT), lambda i_n, i_b: (i_b, i_n)),
            pl.BlockSpec((_BT, _C), lambda i_n, i_b: (i_b, 0)),
        ],
        out_shape=[
            jax.ShapeDtypeStruct((_B, _N), jnp.int8),
            jax.ShapeDtypeStruct((_B, _C), jnp.float32),
        ],
        scratch_shapes=[
            pltpu.VMEM((_NT, _CE), jnp.bfloat16),
            pltpu.VMEM((_BP, _CE), jnp.float32),
        ],
    )(boxes_p, coords_q, features)
    return (box_features, is_inside.view(jnp.bool_))


# speedup vs baseline: 2.6181x; 1.1415x over previous
import jax
import jax.numpy as jnp
from jax.experimental import pallas as pl
from jax.experimental.pallas import tpu as pltpu

_B = 5000
_N = 20000
_C = 256
_CE = 384
_BT = 1024
_NT = 4096
_NB = 5
_NN = 5
_BP = _BT * _NB
_NP = _NT * _NN
_SCALE = 32768.0
_SHIFT = 1 << 24


def _roi_body(boxes_ref, coords_ref, feat_ref, mask_ref, bf_ref,
              fhe_ref, acc_ref):
    i_n = pl.program_id(0)
    i_b = pl.program_id(1)

    @pl.when(i_b == 0)
    def _stage_features():
        fhe_ref[:, :_C] = feat_ref[...].astype(jnp.bfloat16)
        fhe_ref[:, _C:] = jnp.concatenate(
            [jnp.ones((_NT, 1), jnp.bfloat16),
             jnp.zeros((_NT, _CE - _C - 1), jnp.bfloat16)], axis=1)

    @pl.when((i_b == 0) & (i_n == _NN - 1))
    def _zero_tail():
        fhe_ref[pl.ds(_N - (_NN - 1) * _NT, _NN * _NT - _N), :] = jnp.zeros(
            (_NN * _NT - _N, _CE), jnp.bfloat16)

    xq = coords_ref[0:1, :]
    yq = coords_ref[1:2, :]
    x0 = boxes_ref[:, 0:1]
    wx = boxes_ref[:, 1:2]
    y0 = boxes_ref[:, 2:3]
    wy = boxes_ref[:, 3:4]

    cx = (xq - x0) <= wx
    cy = (yq - y0) <= wy
    mi = jnp.where(cx, jnp.where(cy, jnp.int32(1), jnp.int32(0)),
                   jnp.int32(0))
    mask_ref[...] = mi.astype(jnp.int8)
    m = mi.astype(jnp.bfloat16)
    part = jnp.dot(m, fhe_ref[...], preferred_element_type=jnp.float32)

    rows = pl.ds(i_b * _BT, _BT)

    @pl.when(i_n == 0)
    def _init():
        acc_ref[rows, :] = part

    @pl.when(i_n > 0)
    def _acc():
        acc_ref[rows, :] = acc_ref[rows, :] + part

    @pl.when(i_n == _NN - 1)
    def _fin():
        tot = acc_ref[rows, :_C]
        c = jnp.maximum(acc_ref[rows, _C:_C + 1], 1.0)
        bf_ref[...] = tot / c


def kernel(coords, features, bbox_tensor, bbox_sample_association):
    f32, i32 = jnp.float32, jnp.int32

    xk = (coords[:, 0] * _SCALE).astype(i32)
    yk = (coords[:, 1] * _SCALE).astype(i32)
    sk = coords[:, 2].astype(i32)
    xq = xk + sk * _SHIFT
    a = bbox_sample_association.astype(i32)
    x0 = jnp.ceil(bbox_tensor[:, 0, 0] * _SCALE).astype(i32) + a * _SHIFT
    y0 = jnp.ceil(bbox_tensor[:, 0, 1] * _SCALE).astype(i32)
    x1 = (jnp.minimum(jnp.ceil(bbox_tensor[:, 1, 0] * _SCALE).astype(i32) - 1,
                      _SHIFT - 1) + a * _SHIFT)
    y1 = jnp.ceil(bbox_tensor[:, 1, 1] * _SCALE).astype(i32) - 1

    wx = x1 - x0
    wy = y1 - y0
    boxes_p = jnp.pad(jnp.stack([x0, wx, y0, wy], axis=-1),
                      ((0, _BP - _B), (0, 8 - 4)),
                      constant_values=0)
    boxes_p = boxes_p.at[_B:, 0].set(1 << 30)
    boxes_p = boxes_p.at[_B:, 2].set(1 << 30)
    boxes_p = boxes_p.astype(jnp.uint32)
    coords_q = jnp.pad(jnp.stack([xq, yk], axis=0),
                       ((0, 8 - 2), (0, _NP - _N)),
                       constant_values=-(1 << 31)).astype(jnp.uint32)

    grid = (_NN, _NB)
    is_inside, box_features = pl.pallas_call(
        _roi_body,
        grid=grid,
        in_specs=[
            pl.BlockSpec((_BT, 8), lambda i_n, i_b: (i_b, 0)),
            pl.BlockSpec((8, _NT), lambda i_n, i_b: (0, i_n)),
            pl.BlockSpec((_NT, _C), lambda i_n, i_b: (i_n, 0)),
        ],
        out_specs=[
            pl.BlockSpec((_BT, _NT), lambda i_n, i_b: (i_b, i_n)),
            pl.BlockSpec((_BT, _C), lambda i_n, i_b: (i_b, 0)),
        ],
        out_shape=[
            jax.ShapeDtypeStruct((_B, _N), jnp.int8),
            jax.ShapeDtypeStruct((_B, _C), jnp.float32),
        ],
        scratch_shapes=[
            pltpu.VMEM((_NT, _CE), jnp.bfloat16),
            pltpu.VMEM((_BP, _CE), jnp.float32),
        ],
    )(boxes_p, coords_q, features)
    return (box_features, is_inside.view(jnp.bool_))
